# baseline (speedup 1.0000x reference)
import functools

import jax
import jax.numpy as jnp
from jax import lax
from jax.experimental import pallas as pl
from jax.experimental.pallas import tpu as pltpu
from jax.experimental.pallas import tpu_sc as plsc

VOCAB = 50265
HID = 768
MAXPOS = 2050
MAXPOS_PAD = 2176
TYPEV = 2
PAD = 1
EPS = 1e-12
B = 4
S = 2048
N = B * S

_NC = 2
_NS = 16
_NW = _NC * _NS
_B_PER_W = N // _NW
_CH = 128


def _posid_body(ids_ref, out_ref):
    ids = ids_ref[...]
    mask = (ids != PAD)
    maskb = mask.astype(jnp.bfloat16)
    ri = lax.broadcasted_iota(jnp.int32, (S, S), 0)
    ci = lax.broadcasted_iota(jnp.int32, (S, S), 1)
    tri = (ri <= ci).astype(jnp.bfloat16)
    inc = lax.dot_general(maskb, tri, (((1,), (0,)), ((), ())),
                          preferred_element_type=jnp.float32)
    out_ref[...] = inc.astype(jnp.int32) * mask.astype(jnp.int32) + PAD


_posid_call = pl.pallas_call(
    _posid_body,
    out_shape=jax.ShapeDtypeStruct((B, S), jnp.int32),
)


_PMM_BLK = 512
_PMM_GRID = N // _PMM_BLK


def _posmm_body(pid_ref, tab_ref, o_ref):
    pid = pid_ref[0, 0, :]
    cols = lax.broadcasted_iota(jnp.int32, (_PMM_BLK, MAXPOS_PAD), 1)
    onehot = (pid[:, None] == cols).astype(jnp.bfloat16)
    o_ref[...] = lax.dot_general(onehot, tab_ref[...], (((1,), (0,)), ((), ())),
                                 preferred_element_type=jnp.float32)


_posmm_call = pl.pallas_call(
    _posmm_body,
    grid=(_PMM_GRID,),
    in_specs=[
        pl.BlockSpec((1, 1, _PMM_BLK), lambda i: (i, 0, 0)),
        pl.BlockSpec((MAXPOS_PAD, HID), lambda i: (0, 0)),
    ],
    out_specs=pl.BlockSpec((_PMM_BLK, HID), lambda i: (i, 0)),
    out_shape=jax.ShapeDtypeStruct((N, HID), jnp.float32),
)


def _gather_body(word_hbm, wid_hbm, ow_hbm, idw_v, rw_v, semw):
    w = lax.axis_index("s") * _NC + lax.axis_index("c")
    base = w * _B_PER_W

    @pl.loop(0, _B_PER_W, step=_CH)
    def _(c):
        off = base + c
        pltpu.sync_copy(wid_hbm.at[pl.ds(off, _CH)], idw_v)
        pltpu.async_copy(word_hbm.at[idw_v], rw_v, semw).wait()
        pltpu.sync_copy(rw_v, ow_hbm.at[pl.ds(off, _CH)])


@functools.cache
def _gather_call():
    return functools.partial(
        pl.kernel,
        out_type=jax.ShapeDtypeStruct((N, HID), jnp.float32),
        mesh=plsc.VectorSubcoreMesh(core_axis_name="c", subcore_axis_name="s"),
        scratch_types=[
            pltpu.VMEM((_CH,), jnp.int32),
            pltpu.VMEM((_CH, HID), jnp.float32),
            pltpu.SemaphoreType.DMA,
        ],
    )(_gather_body)


_LN_BLK = 1024


def _ln_body(gw_ref, gp_ref, type_ref, g_ref, b_ref, o_ref):
    x = gw_ref[...] + gp_ref[...] + type_ref[0, :][None, :]
    mean = jnp.mean(x, axis=-1, keepdims=True)
    xc = x - mean
    var = jnp.mean(xc * xc, axis=-1, keepdims=True)
    o_ref[...] = (xc * lax.rsqrt(var + EPS)) * g_ref[0, :][None, :] \
        + b_ref[0, :][None, :]


_ln_call = pl.pallas_call(
    _ln_body,
    grid=(N // _LN_BLK,),
    in_specs=[
        pl.BlockSpec((_LN_BLK, HID), lambda i: (i, 0)),
        pl.BlockSpec((_LN_BLK, HID), lambda i: (i, 0)),
        pl.BlockSpec((TYPEV, HID), lambda i: (0, 0)),
        pl.BlockSpec((1, HID), lambda i: (0, 0)),
        pl.BlockSpec((1, HID), lambda i: (0, 0)),
    ],
    out_specs=pl.BlockSpec((_LN_BLK, HID), lambda i: (i, 0)),
    out_shape=jax.ShapeDtypeStruct((N, HID), jnp.float32),
)


def kernel(input_ids, word_emb, pos_emb, type_emb, ln_gamma, ln_beta):
    position_ids = _posid_call(input_ids)
    gw = _gather_call()(word_emb, input_ids.reshape(N))
    tab = jnp.pad(pos_emb, ((0, MAXPOS_PAD - MAXPOS), (0, 0))) \
        .astype(jnp.bfloat16)
    pose = _posmm_call(position_ids.reshape(_PMM_GRID, 1, _PMM_BLK), tab)
    out = _ln_call(gw, pose, type_emb,
                   ln_gamma.reshape(1, HID), ln_beta.reshape(1, HID))
    return out.reshape(B, S, HID), position_ids


# speedup vs baseline: 1.0461x; 1.0461x over previous
import functools

import jax
import jax.numpy as jnp
from jax import lax
from jax.experimental import pallas as pl
from jax.experimental.pallas import tpu as pltpu
from jax.experimental.pallas import tpu_sc as plsc

VOCAB = 50265
HID = 768
MAXPOS = 2050
TYPEV = 2
PAD = 1
EPS = 1e-12
B = 4
S = 2048
N = B * S

_NC = 2
_NS = 16
_NW = _NC * _NS
_B_PER_W = N // _NW
_CH = 64


def _posid_body(ids_ref, out_ref):
    ids = ids_ref[...]
    mask = (ids != PAD)
    maskb = mask.astype(jnp.bfloat16)
    ri = lax.broadcasted_iota(jnp.int32, (S, S), 0)
    ci = lax.broadcasted_iota(jnp.int32, (S, S), 1)
    tri = (ri <= ci).astype(jnp.bfloat16)
    inc = lax.dot_general(maskb, tri, (((1,), (0,)), ((), ())),
                          preferred_element_type=jnp.float32)
    out_ref[...] = inc.astype(jnp.int32) * mask.astype(jnp.int32) + PAD


_posid_call = pl.pallas_call(
    _posid_body,
    out_shape=jax.ShapeDtypeStruct((B, S), jnp.int32),
)


def _gather_body(word_hbm, pos_hbm, wid_hbm, pid_hbm, ow_hbm, op_hbm,
                 idx0, idx1, rows0, rows1, gsem0, gsem1, wsem0, wsem1):
    w = lax.axis_index("s") * _NC + lax.axis_index("c")
    base = w * _B_PER_W

    n_ch = _B_PER_W // _CH
    items = [(word_hbm, wid_hbm, ow_hbm, c * _CH) for c in range(n_ch)]
    items += [(pos_hbm, pid_hbm, op_hbm, c * _CH) for c in range(n_ch)]

    idx_bufs = (idx0, idx1)
    row_bufs = (rows0, rows1)
    gsems = (gsem0, gsem1)
    wsems = (wsem0, wsem1)

    writebacks = []
    for k, (tab, idx_hbm, out_hbm, coff) in enumerate(items):
        b = k % 2
        if k >= 2:
            writebacks[k - 2].wait()
        off = base + coff
        pltpu.sync_copy(idx_hbm.at[pl.ds(off, _CH)], idx_bufs[b])
        pltpu.async_copy(tab.at[idx_bufs[b]], row_bufs[b], gsems[b]).wait()
        writebacks.append(
            pltpu.async_copy(row_bufs[b], out_hbm.at[pl.ds(off, _CH)],
                             wsems[b]))
    writebacks[-2].wait()
    writebacks[-1].wait()


@functools.cache
def _gather_call():
    return functools.partial(
        pl.kernel,
        out_type=(jax.ShapeDtypeStruct((N, HID), jnp.float32),
                  jax.ShapeDtypeStruct((N, HID), jnp.float32)),
        mesh=plsc.VectorSubcoreMesh(core_axis_name="c", subcore_axis_name="s"),
        scratch_types=[
            pltpu.VMEM((_CH,), jnp.int32),
            pltpu.VMEM((_CH,), jnp.int32),
            pltpu.VMEM((_CH, HID), jnp.float32),
            pltpu.VMEM((_CH, HID), jnp.float32),
            pltpu.SemaphoreType.DMA,
            pltpu.SemaphoreType.DMA,
            pltpu.SemaphoreType.DMA,
            pltpu.SemaphoreType.DMA,
        ],
    )(_gather_body)


_LN_BLK = 1024


def _ln_body(gw_ref, gp_ref, type_ref, g_ref, b_ref, o_ref):
    x = gw_ref[...] + gp_ref[...] + type_ref[0, :][None, :]
    mean = jnp.mean(x, axis=-1, keepdims=True)
    xc = x - mean
    var = jnp.mean(xc * xc, axis=-1, keepdims=True)
    o_ref[...] = (xc * lax.rsqrt(var + EPS)) * g_ref[0, :][None, :] \
        + b_ref[0, :][None, :]


_ln_call = pl.pallas_call(
    _ln_body,
    grid=(N // _LN_BLK,),
    in_specs=[
        pl.BlockSpec((_LN_BLK, HID), lambda i: (i, 0)),
        pl.BlockSpec((_LN_BLK, HID), lambda i: (i, 0)),
        pl.BlockSpec((TYPEV, HID), lambda i: (0, 0)),
        pl.BlockSpec((1, HID), lambda i: (0, 0)),
        pl.BlockSpec((1, HID), lambda i: (0, 0)),
    ],
    out_specs=pl.BlockSpec((_LN_BLK, HID), lambda i: (i, 0)),
    out_shape=jax.ShapeDtypeStruct((N, HID), jnp.float32),
)


def kernel(input_ids, word_emb, pos_emb, type_emb, ln_gamma, ln_beta):
    position_ids = _posid_call(input_ids)
    gw, gp = _gather_call()(word_emb, pos_emb,
                            input_ids.reshape(N), position_ids.reshape(N))
    out = _ln_call(gw, gp, type_emb,
                   ln_gamma.reshape(1, HID), ln_beta.reshape(1, HID))
    return out.reshape(B, S, HID), position_ids


# speedup vs baseline: 1.1530x; 1.1023x over previous
import functools

import jax
import jax.numpy as jnp
from jax import lax
from jax.experimental import pallas as pl
from jax.experimental.pallas import tpu as pltpu
from jax.experimental.pallas import tpu_sc as plsc

VOCAB = 50265
HID = 768
MAXPOS = 2050
TYPEV = 2
PAD = 1
EPS = 1e-12
B = 4
S = 2048
N = B * S

_SLAB = 256
_NSLAB = 5
_TAB_BLKS = 9
_HALF = S // 2

_NC = 2
_NS = 16
_NW = _NC * _NS
_B_PER_W = N // _NW
_CH = 64


def _posid_body(ids_ref, out_ref, ss_ref):
    ids = ids_ref[...]
    mask = (ids != PAD)
    maskb = mask.astype(jnp.bfloat16)
    ri = lax.broadcasted_iota(jnp.int32, (S, S), 0)
    ci = lax.broadcasted_iota(jnp.int32, (S, S), 1)
    tri = (ri <= ci).astype(jnp.bfloat16)
    inc = lax.dot_general(maskb, tri, (((1,), (0,)), ((), ())),
                          preferred_element_type=jnp.float32)
    inc = inc.astype(jnp.int32)
    out_ref[...] = inc * mask.astype(jnp.int32) + PAD
    ss_ref[...] = (inc[:, _HALF - 1:_HALF] + 1) // _SLAB


_posid_call = pl.pallas_call(
    _posid_body,
    out_shape=(jax.ShapeDtypeStruct((B, S), jnp.int32),
               jax.ShapeDtypeStruct((B, 1), jnp.int32)),
)


def _gather_body(word_hbm, wid_hbm, ow_hbm,
                 idx0, idx1, rows0, rows1, gsem0, gsem1, wsem0, wsem1):
    w = lax.axis_index("s") * _NC + lax.axis_index("c")
    base = w * _B_PER_W

    idx_bufs = (idx0, idx1)
    row_bufs = (rows0, rows1)
    gsems = (gsem0, gsem1)
    wsems = (wsem0, wsem1)

    n_ch = _B_PER_W // _CH
    writebacks = []
    for k in range(n_ch):
        b = k % 2
        if k >= 2:
            writebacks[k - 2].wait()
        off = base + k * _CH
        pltpu.sync_copy(wid_hbm.at[pl.ds(off, _CH)], idx_bufs[b])
        pltpu.async_copy(word_hbm.at[idx_bufs[b]], row_bufs[b], gsems[b]).wait()
        writebacks.append(
            pltpu.async_copy(row_bufs[b], ow_hbm.at[pl.ds(off, _CH)],
                             wsems[b]))
    writebacks[-2].wait()
    writebacks[-1].wait()


@functools.cache
def _gather_call():
    return functools.partial(
        pl.kernel,
        out_type=jax.ShapeDtypeStruct((N, HID), jnp.float32),
        mesh=plsc.VectorSubcoreMesh(core_axis_name="c", subcore_axis_name="s"),
        scratch_types=[
            pltpu.VMEM((_CH,), jnp.int32),
            pltpu.VMEM((_CH,), jnp.int32),
            pltpu.VMEM((_CH, HID), jnp.float32),
            pltpu.VMEM((_CH, HID), jnp.float32),
            pltpu.SemaphoreType.DMA,
            pltpu.SemaphoreType.DMA,
            pltpu.SemaphoreType.DMA,
            pltpu.SemaphoreType.DMA,
        ],
    )(_gather_body)


def _ln_body(ss_ref, pid_ref, gw_ref, s0, s1, s2, s3, s4,
             p1_ref, type_ref, g_ref, b_ref, o_ref):
    h = pl.program_id(0)
    b = pl.program_id(1)
    ssv = h * ss_ref[b, 0]
    pidc = pid_ref[...]
    nonpad = pidc != PAD

    acc = jnp.zeros((_HALF, HID), dtype=jnp.float32)
    for j, sref in enumerate((s0, s1, s2, s3, s4)):
        cols = (ssv + j) * _SLAB + lax.broadcasted_iota(
            jnp.int32, (_HALF, _SLAB), 1)
        onehot = ((pidc == cols) & nonpad).astype(jnp.bfloat16)
        acc = acc + lax.dot_general(
            onehot, sref[0].astype(jnp.bfloat16), (((1,), (0,)), ((), ())),
            preferred_element_type=jnp.float32)

    padrow = (~nonpad).astype(jnp.float32) * p1_ref[...]
    x = gw_ref[...] + acc + padrow + type_ref[0, :][None, :]
    mean = jnp.mean(x, axis=-1, keepdims=True)
    xc = x - mean
    var = jnp.mean(xc * xc, axis=-1, keepdims=True)
    o_ref[...] = (xc * lax.rsqrt(var + EPS)) * g_ref[0, :][None, :] \
        + b_ref[0, :][None, :]


def _slab_spec(j):
    return pl.BlockSpec((1, _SLAB, HID),
                        lambda h, b, ss, j=j: (h * ss[b, 0] + j, 0, 0))


_ln_call = pl.pallas_call(
    _ln_body,
    grid_spec=pltpu.PrefetchScalarGridSpec(
        num_scalar_prefetch=1,
        grid=(2, B),
        in_specs=[
            pl.BlockSpec((_HALF, 1), lambda h, b, ss: (2 * b + h, 0)),
            pl.BlockSpec((_HALF, HID), lambda h, b, ss: (2 * b + h, 0)),
            _slab_spec(0), _slab_spec(1), _slab_spec(2),
            _slab_spec(3), _slab_spec(4),
            pl.BlockSpec((1, HID), lambda h, b, ss: (0, 0)),
            pl.BlockSpec((TYPEV, HID), lambda h, b, ss: (0, 0)),
            pl.BlockSpec((1, HID), lambda h, b, ss: (0, 0)),
            pl.BlockSpec((1, HID), lambda h, b, ss: (0, 0)),
        ],
        out_specs=pl.BlockSpec((_HALF, HID), lambda h, b, ss: (2 * b + h, 0)),
    ),
    out_shape=jax.ShapeDtypeStruct((N, HID), jnp.float32),
)


def kernel(input_ids, word_emb, pos_emb, type_emb, ln_gamma, ln_beta):
    position_ids, ss = _posid_call(input_ids)
    gw = _gather_call()(word_emb, input_ids.reshape(N))
    tab = jnp.pad(pos_emb, ((0, _TAB_BLKS * _SLAB - MAXPOS), (0, 0))) \
        .reshape(_TAB_BLKS, _SLAB, HID)
    out = _ln_call(ss, position_ids.reshape(N, 1), gw,
                   tab, tab, tab, tab, tab,
                   pos_emb[PAD:PAD + 1], type_emb,
                   ln_gamma.reshape(1, HID), ln_beta.reshape(1, HID))
    return out.reshape(B, S, HID), position_ids


# speedup vs baseline: 1.2360x; 1.0720x over previous
import functools

import jax
import jax.numpy as jnp
from jax import lax
from jax.experimental import pallas as pl
from jax.experimental.pallas import tpu as pltpu
from jax.experimental.pallas import tpu_sc as plsc

VOCAB = 50265
HID = 768
MAXPOS = 2050
TYPEV = 2
PAD = 1
EPS = 1e-12
B = 4
S = 2048
N = B * S

_SLAB = 256
_KSLAB = 1280
_TAB_PAD = 2304
_HALF = S // 2

_NC = 2
_NS = 16
_NW = _NC * _NS
_B_PER_W = N // _NW
_CH = 128


def _posid_body(ids_ref, ptab_ref, out_ref, ss_ref, tab_ref):
    ids = ids_ref[...]
    mask = (ids != PAD)
    maskb = mask.astype(jnp.bfloat16)
    ri = lax.broadcasted_iota(jnp.int32, (S, S), 0)
    ci = lax.broadcasted_iota(jnp.int32, (S, S), 1)
    tri = (ri <= ci).astype(jnp.bfloat16)
    inc = lax.dot_general(maskb, tri, (((1,), (0,)), ((), ())),
                          preferred_element_type=jnp.float32)
    inc = inc.astype(jnp.int32)
    out_ref[...] = inc * mask.astype(jnp.int32) + PAD
    ss_ref[...] = (inc[:, _HALF - 1:_HALF] + 1) // _SLAB
    t = ptab_ref[...]
    tab_ref[pl.ds(0, 2048), :] = t[0:2048, :].astype(jnp.bfloat16)
    tail = jnp.concatenate(
        [t[2048:MAXPOS, :], jnp.zeros((16 - (MAXPOS - 2048), HID),
                                      jnp.float32)], axis=0)
    tab_ref[pl.ds(2048, 16), :] = tail.astype(jnp.bfloat16)
    tab_ref[pl.ds(2064, _TAB_PAD - 2064), :] = jnp.zeros(
        (_TAB_PAD - 2064, HID), jnp.bfloat16)


_posid_call = pl.pallas_call(
    _posid_body,
    out_shape=(jax.ShapeDtypeStruct((B, S), jnp.int32),
               jax.ShapeDtypeStruct((B, 1), jnp.int32),
               jax.ShapeDtypeStruct((_TAB_PAD, HID), jnp.bfloat16)),
)


def _gather_body(word_hbm, wid_hbm, ow_hbm, idx_v, rows_v, sem):
    w = lax.axis_index("s") * _NC + lax.axis_index("c")
    base = w * _B_PER_W

    @pl.loop(0, _B_PER_W, step=_CH)
    def _(c):
        off = base + c
        pltpu.sync_copy(wid_hbm.at[pl.ds(off, _CH)], idx_v)
        pltpu.async_copy(word_hbm.at[idx_v], rows_v, sem).wait()
        pltpu.sync_copy(rows_v, ow_hbm.at[pl.ds(off, _CH)])


@functools.cache
def _gather_call():
    return functools.partial(
        pl.kernel,
        out_type=jax.ShapeDtypeStruct((N, HID), jnp.float32),
        mesh=plsc.VectorSubcoreMesh(core_axis_name="c", subcore_axis_name="s"),
        scratch_types=[
            pltpu.VMEM((_CH,), jnp.int32),
            pltpu.VMEM((_CH, HID), jnp.float32),
            pltpu.SemaphoreType.DMA,
        ],
    )(_gather_body)


def _ln_body(ss_ref, pid_ref, gw_ref, tab_ref, type_ref, g_ref, b_ref,
             o_ref):
    h = pl.program_id(0)
    b = pl.program_id(1)
    ssv = h * ss_ref[b, 0]
    pidc = pid_ref[...]
    nonpad = pidc != PAD

    slab = tab_ref[pl.ds(ssv * _SLAB, _KSLAB), :]
    cols = ssv * _SLAB + lax.broadcasted_iota(jnp.int32, (_HALF, _KSLAB), 1)
    onehot = ((pidc == cols) & nonpad).astype(jnp.bfloat16)
    acc = lax.dot_general(onehot, slab, (((1,), (0,)), ((), ())),
                          preferred_element_type=jnp.float32)

    padrow = (~nonpad).astype(jnp.float32) \
        * tab_ref[PAD, :][None, :].astype(jnp.float32)
    x = gw_ref[...] + acc + padrow + type_ref[0, :][None, :]
    mean = jnp.mean(x, axis=-1, keepdims=True)
    xc = x - mean
    var = jnp.mean(xc * xc, axis=-1, keepdims=True)
    o_ref[...] = (xc * lax.rsqrt(var + EPS)) * g_ref[0, :][None, :] \
        + b_ref[0, :][None, :]


_ln_call = pl.pallas_call(
    _ln_body,
    grid_spec=pltpu.PrefetchScalarGridSpec(
        num_scalar_prefetch=1,
        grid=(2, B),
        in_specs=[
            pl.BlockSpec((_HALF, 1), lambda h, b, ss: (2 * b + h, 0)),
            pl.BlockSpec((_HALF, HID), lambda h, b, ss: (2 * b + h, 0)),
            pl.BlockSpec((_TAB_PAD, HID), lambda h, b, ss: (0, 0)),
            pl.BlockSpec((TYPEV, HID), lambda h, b, ss: (0, 0)),
            pl.BlockSpec((1, HID), lambda h, b, ss: (0, 0)),
            pl.BlockSpec((1, HID), lambda h, b, ss: (0, 0)),
        ],
        out_specs=pl.BlockSpec((_HALF, HID), lambda h, b, ss: (2 * b + h, 0)),
    ),
    out_shape=jax.ShapeDtypeStruct((N, HID), jnp.float32),
)


def kernel(input_ids, word_emb, pos_emb, type_emb, ln_gamma, ln_beta):
    position_ids, ss, tab = _posid_call(input_ids, pos_emb)
    gw = _gather_call()(word_emb, input_ids.reshape(N))
    out = _ln_call(ss, position_ids.reshape(N, 1), gw, tab, type_emb,
                   ln_gamma.reshape(1, HID), ln_beta.reshape(1, HID))
    return out.reshape(B, S, HID), position_ids


# speedup vs baseline: 1.2870x; 1.0412x over previous
import functools

import jax
import jax.numpy as jnp
from jax import lax
from jax.experimental import pallas as pl
from jax.experimental.pallas import tpu as pltpu
from jax.experimental.pallas import tpu_sc as plsc

VOCAB = 50265
HID = 768
MAXPOS = 2050
TYPEV = 2
PAD = 1
EPS = 1e-12
B = 4
S = 2048
N = B * S

_SLAB = 256
_BLK = 512
_NBLK = S // _BLK
_KSLAB = _BLK + _SLAB
_TAB_PAD = 2304

_NC = 2
_NS = 16
_NW = _NC * _NS
_B_PER_W = N // _NW
_CH = 128


def _posid_body(ids_ref, ptab_ref, out_ref, ss_ref, tab_ref):
    ids = ids_ref[...]
    mask = (ids != PAD)
    maskb = mask.astype(jnp.bfloat16)
    ri = lax.broadcasted_iota(jnp.int32, (S, S), 0)
    ci = lax.broadcasted_iota(jnp.int32, (S, S), 1)
    tri = (ri <= ci).astype(jnp.bfloat16)
    inc = lax.dot_general(maskb, tri, (((1,), (0,)), ((), ())),
                          preferred_element_type=jnp.float32)
    inc = inc.astype(jnp.int32)
    out_ref[...] = inc * mask.astype(jnp.int32) + PAD
    ss_cols = [jnp.zeros((B, 1), jnp.int32)]
    for k in range(1, _NBLK):
        ss_cols.append((inc[:, k * _BLK - 1:k * _BLK] + 1) // _SLAB)
    ss_ref[...] = jnp.concatenate(ss_cols, axis=1)
    t = ptab_ref[...]
    tab_ref[pl.ds(0, 2048), :] = t[0:2048, :].astype(jnp.bfloat16)
    tail = jnp.concatenate(
        [t[2048:MAXPOS, :], jnp.zeros((16 - (MAXPOS - 2048), HID),
                                      jnp.float32)], axis=0)
    tab_ref[pl.ds(2048, 16), :] = tail.astype(jnp.bfloat16)
    tab_ref[pl.ds(2064, _TAB_PAD - 2064), :] = jnp.zeros(
        (_TAB_PAD - 2064, HID), jnp.bfloat16)


_posid_call = pl.pallas_call(
    _posid_body,
    out_shape=(jax.ShapeDtypeStruct((B, S), jnp.int32),
               jax.ShapeDtypeStruct((B, _NBLK), jnp.int32),
               jax.ShapeDtypeStruct((_TAB_PAD, HID), jnp.bfloat16)),
)


def _gather_body(word_hbm, wid_hbm, ow_hbm, idx_v, rows_v, sem):
    w = lax.axis_index("s") * _NC + lax.axis_index("c")
    base = w * _B_PER_W

    @pl.loop(0, _B_PER_W, step=_CH)
    def _(c):
        off = base + c
        pltpu.sync_copy(wid_hbm.at[pl.ds(off, _CH)], idx_v)
        pltpu.async_copy(word_hbm.at[idx_v], rows_v, sem).wait()
        pltpu.sync_copy(rows_v, ow_hbm.at[pl.ds(off, _CH)])


@functools.cache
def _gather_call():
    return functools.partial(
        pl.kernel,
        out_type=jax.ShapeDtypeStruct((N, HID), jnp.float32),
        mesh=plsc.VectorSubcoreMesh(core_axis_name="c", subcore_axis_name="s"),
        scratch_types=[
            pltpu.VMEM((_CH,), jnp.int32),
            pltpu.VMEM((_CH, HID), jnp.float32),
            pltpu.SemaphoreType.DMA,
        ],
    )(_gather_body)


def _ln_body(ss_ref, pid_ref, gw_ref, tab_ref, type_ref, g_ref, b_ref,
             o_ref):
    k = pl.program_id(0)
    b = pl.program_id(1)
    ssv = ss_ref[b, k]
    pidc = pid_ref[...]

    slab = tab_ref[pl.ds(ssv * _SLAB, _KSLAB), :]
    lpid = pidc - ssv * _SLAB
    cols = lax.broadcasted_iota(jnp.int32, (_BLK, _KSLAB), 1)
    onehot = (lpid == cols).astype(jnp.bfloat16)
    acc = lax.dot_general(onehot, slab, (((1,), (0,)), ((), ())),
                          preferred_element_type=jnp.float32)

    padgate = jnp.where(ssv > 0, 1.0, 0.0)
    padrow = (padgate * (pidc == PAD).astype(jnp.float32)) \
        * tab_ref[PAD, :][None, :].astype(jnp.float32)
    x = gw_ref[...] + acc + padrow + type_ref[0, :][None, :]
    mean = jnp.mean(x, axis=-1, keepdims=True)
    xc = x - mean
    var = jnp.mean(xc * xc, axis=-1, keepdims=True)
    o_ref[...] = (xc * lax.rsqrt(var + EPS)) * g_ref[0, :][None, :] \
        + b_ref[0, :][None, :]


_ln_call = pl.pallas_call(
    _ln_body,
    grid_spec=pltpu.PrefetchScalarGridSpec(
        num_scalar_prefetch=1,
        grid=(_NBLK, B),
        in_specs=[
            pl.BlockSpec((_BLK, 1), lambda k, b, ss: (_NBLK * b + k, 0)),
            pl.BlockSpec((_BLK, HID), lambda k, b, ss: (_NBLK * b + k, 0)),
            pl.BlockSpec((_TAB_PAD, HID), lambda k, b, ss: (0, 0)),
            pl.BlockSpec((TYPEV, HID), lambda k, b, ss: (0, 0)),
            pl.BlockSpec((1, HID), lambda k, b, ss: (0, 0)),
            pl.BlockSpec((1, HID), lambda k, b, ss: (0, 0)),
        ],
        out_specs=pl.BlockSpec((_BLK, HID),
                               lambda k, b, ss: (_NBLK * b + k, 0)),
    ),
    out_shape=jax.ShapeDtypeStruct((N, HID), jnp.float32),
)


def kernel(input_ids, word_emb, pos_emb, type_emb, ln_gamma, ln_beta):
    position_ids, ss, tab = _posid_call(input_ids, pos_emb)
    gw = _gather_call()(word_emb, input_ids.reshape(N))
    out = _ln_call(ss, position_ids.reshape(N, 1), gw, tab, type_emb,
                   ln_gamma.reshape(1, HID), ln_beta.reshape(1, HID))
    return out.reshape(B, S, HID), position_ids


# speedup vs baseline: 1.3143x; 1.0212x over previous
import functools

import jax
import jax.numpy as jnp
from jax import lax
from jax.experimental import pallas as pl
from jax.experimental.pallas import tpu as pltpu
from jax.experimental.pallas import tpu_sc as plsc

VOCAB = 50265
HID = 768
MAXPOS = 2050
TYPEV = 2
PAD = 1
EPS = 1e-12
B = 4
S = 2048
N = B * S
_NH = N // 2

_SLAB = 256
_BLK = 512
_NBLK = S // _BLK
_KSLAB = _BLK + _SLAB
_TAB_PAD = 2304

_NC = 2
_NS = 16
_NW = _NC * _NS
_B_PER_W = _NH // _NW
_CH = 128


def _posid_body(ids_ref, ptab_ref, out_ref, ss_ref, tab_ref):
    ids = ids_ref[...]
    mask = (ids != PAD)
    maskb = mask.astype(jnp.bfloat16)
    ri = lax.broadcasted_iota(jnp.int32, (S, S), 0)
    ci = lax.broadcasted_iota(jnp.int32, (S, S), 1)
    tri = (ri <= ci).astype(jnp.bfloat16)
    inc = lax.dot_general(maskb, tri, (((1,), (0,)), ((), ())),
                          preferred_element_type=jnp.float32)
    inc = inc.astype(jnp.int32)
    out_ref[...] = inc * mask.astype(jnp.int32) + PAD
    ss_cols = [jnp.zeros((B, 1), jnp.int32)]
    for k in range(1, _NBLK):
        ss_cols.append((inc[:, k * _BLK - 1:k * _BLK] + 1) // _SLAB)
    ss_ref[...] = jnp.concatenate(ss_cols, axis=1)
    t = ptab_ref[...]
    tab_ref[pl.ds(0, 2048), :] = t[0:2048, :].astype(jnp.bfloat16)
    tail = jnp.concatenate(
        [t[2048:MAXPOS, :], jnp.zeros((16 - (MAXPOS - 2048), HID),
                                      jnp.float32)], axis=0)
    tab_ref[pl.ds(2048, 16), :] = tail.astype(jnp.bfloat16)
    tab_ref[pl.ds(2064, _TAB_PAD - 2064), :] = jnp.zeros(
        (_TAB_PAD - 2064, HID), jnp.bfloat16)


_posid_call = pl.pallas_call(
    _posid_body,
    out_shape=(jax.ShapeDtypeStruct((B, S), jnp.int32),
               jax.ShapeDtypeStruct((B, _NBLK), jnp.int32),
               jax.ShapeDtypeStruct((_TAB_PAD, HID), jnp.bfloat16)),
)


def _gather_body(word_hbm, wid_hbm, ow_hbm, idx_v, rows_v, sem):
    w = lax.axis_index("s") * _NC + lax.axis_index("c")
    base = w * _B_PER_W

    @pl.loop(0, _B_PER_W, step=_CH)
    def _(c):
        off = base + c
        pltpu.sync_copy(wid_hbm.at[pl.ds(off, _CH)], idx_v)
        pltpu.async_copy(word_hbm.at[idx_v], rows_v, sem).wait()
        pltpu.sync_copy(rows_v, ow_hbm.at[pl.ds(off, _CH)])


@functools.cache
def _gather_call():
    return functools.partial(
        pl.kernel,
        out_type=jax.ShapeDtypeStruct((_NH, HID), jnp.float32),
        mesh=plsc.VectorSubcoreMesh(core_axis_name="c", subcore_axis_name="s"),
        scratch_types=[
            pltpu.VMEM((_CH,), jnp.int32),
            pltpu.VMEM((_CH, HID), jnp.float32),
            pltpu.SemaphoreType.DMA,
        ],
    )(_gather_body)


def _make_ln_body(boff, aliased):
    def _ln_body(*refs):
        if aliased:
            (ss_ref, pid_ref, gw_ref, tab_ref, type_ref, g_ref, b_ref,
             _prev_ref, o_ref) = refs
        else:
            (ss_ref, pid_ref, gw_ref, tab_ref, type_ref, g_ref, b_ref,
             o_ref) = refs
        k = pl.program_id(0)
        b = pl.program_id(1) + boff
        ssv = ss_ref[b, k]
        pidc = pid_ref[...]

        slab = tab_ref[pl.ds(ssv * _SLAB, _KSLAB), :]
        lpid = pidc - ssv * _SLAB
        cols = lax.broadcasted_iota(jnp.int32, (_BLK, _KSLAB), 1)
        onehot = (lpid == cols).astype(jnp.bfloat16)
        acc = lax.dot_general(onehot, slab, (((1,), (0,)), ((), ())),
                              preferred_element_type=jnp.float32)

        padgate = jnp.where(ssv > 0, 1.0, 0.0)
        padrow = (padgate * (pidc == PAD).astype(jnp.float32)) \
            * tab_ref[PAD, :][None, :].astype(jnp.float32)
        x = gw_ref[...] + acc + padrow + type_ref[0, :][None, :]
        mean = jnp.mean(x, axis=-1, keepdims=True)
        xc = x - mean
        var = jnp.mean(xc * xc, axis=-1, keepdims=True)
        o_ref[...] = (xc * lax.rsqrt(var + EPS)) * g_ref[0, :][None, :] \
            + b_ref[0, :][None, :]
    return _ln_body


def _make_ln_call(boff, aliased):
    def _ob(k, b, ss):
        return (_NBLK * (b + boff) + k, 0)

    in_specs = [
        pl.BlockSpec((_BLK, 1), _ob),
        pl.BlockSpec((_BLK, HID), lambda k, b, ss: (_NBLK * b + k, 0)),
        pl.BlockSpec((_TAB_PAD, HID), lambda k, b, ss: (0, 0)),
        pl.BlockSpec((TYPEV, HID), lambda k, b, ss: (0, 0)),
        pl.BlockSpec((1, HID), lambda k, b, ss: (0, 0)),
        pl.BlockSpec((1, HID), lambda k, b, ss: (0, 0)),
    ]
    kwargs = {}
    if aliased:
        in_specs.append(pl.BlockSpec(memory_space=pl.ANY))
        kwargs["input_output_aliases"] = {7: 0}
    return pl.pallas_call(
        _make_ln_body(boff, aliased),
        grid_spec=pltpu.PrefetchScalarGridSpec(
            num_scalar_prefetch=1,
            grid=(_NBLK, B // 2),
            in_specs=in_specs,
            out_specs=pl.BlockSpec((_BLK, HID), _ob),
        ),
        out_shape=jax.ShapeDtypeStruct((N, HID), jnp.float32),
        **kwargs,
    )


_ln_call_a = _make_ln_call(0, aliased=False)
_ln_call_b = _make_ln_call(B // 2, aliased=True)


def kernel(input_ids, word_emb, pos_emb, type_emb, ln_gamma, ln_beta):
    position_ids, ss, tab = _posid_call(input_ids, pos_emb)
    ids_flat = input_ids.reshape(N)
    pid_col = position_ids.reshape(N, 1)
    gw_a = _gather_call()(word_emb, ids_flat[:_NH])
    gw_b = _gather_call()(word_emb, ids_flat[_NH:])
    g2 = ln_gamma.reshape(1, HID)
    b2 = ln_beta.reshape(1, HID)
    out_a = _ln_call_a(ss, pid_col, gw_a, tab, type_emb, g2, b2)
    out = _ln_call_b(ss, pid_col, gw_b, tab, type_emb, g2, b2, out_a)
    return out.reshape(B, S, HID), position_ids
